# Initial kernel scaffold; baseline (speedup 1.0000x reference)
import jax
import jax.numpy as jnp
from jax.experimental import pallas as pl


def kernel(entity_embedding, relation_embedding, positive_sample, negative_sample, mode):
    raise NotImplementedError("write your pallas kernel here")


# speedup vs baseline: 1.2938x; 1.2938x over previous
import functools

import jax
import jax.numpy as jnp
from jax import lax
from jax.experimental import pallas as pl
from jax.experimental.pallas import tpu as pltpu
from jax.experimental.pallas import tpu_sc as plsc

GAMMA = 12.0
U = 1.0

_NC = 2
_NS = 16
_NW = _NC * _NS
_CHUNK = 128


def _sc_gather(entity_embedding, relation_embedding, neg_idx, head_idx,
               rel_idx, tail_idx):
    n_total = neg_idx.shape[0]
    b = head_idx.shape[0]
    ent_dim = entity_embedding.shape[1]
    rel_dim = relation_embedding.shape[1]
    assert rel_dim == ent_dim
    n_per_w = n_total // _NW
    n_chunks = n_per_w // _CHUNK
    b_per_w = b // _NW

    mesh = plsc.VectorSubcoreMesh(core_axis_name="c", subcore_axis_name="s",
                                  num_cores=_NC, num_subcores=_NS)

    @functools.partial(
        pl.kernel,
        out_type=(
            jax.ShapeDtypeStruct((n_total, ent_dim), jnp.float32),
            jax.ShapeDtypeStruct((b, ent_dim), jnp.float32),
            jax.ShapeDtypeStruct((b, ent_dim), jnp.float32),
            jax.ShapeDtypeStruct((b, ent_dim), jnp.float32),
        ),
        mesh=mesh,
        scratch_types=[
            pltpu.VMEM((n_per_w,), jnp.int32),
            pltpu.VMEM((_CHUNK, ent_dim), jnp.float32),
            pltpu.VMEM((b_per_w,), jnp.int32),
            pltpu.VMEM((b_per_w, ent_dim), jnp.float32),
            pltpu.SemaphoreType.DMA,
        ],
    )
    def k(ent_hbm, rel_hbm, nidx_hbm, hidx_hbm, ridx_hbm, tidx_hbm,
          nout_hbm, hout_hbm, tout_hbm, rout_hbm,
          nidx_v, nrows_v, pidx_v, prows_v, sem):
        wid = lax.axis_index("s") * _NC + lax.axis_index("c")
        nbase = wid * n_per_w
        pbase = wid * b_per_w

        pltpu.sync_copy(nidx_hbm.at[pl.ds(nbase, n_per_w)], nidx_v)

        def body(g, carry):
            off = g * _CHUNK
            pltpu.async_copy(ent_hbm.at[nidx_v.at[pl.ds(off, _CHUNK)]],
                             nrows_v, sem).wait()
            pltpu.sync_copy(nrows_v, nout_hbm.at[pl.ds(nbase + off, _CHUNK)])
            return carry

        lax.fori_loop(0, n_chunks, body, 0, unroll=False)

        pltpu.sync_copy(hidx_hbm.at[pl.ds(pbase, b_per_w)], pidx_v)
        pltpu.async_copy(ent_hbm.at[pidx_v], prows_v, sem).wait()
        pltpu.sync_copy(prows_v, hout_hbm.at[pl.ds(pbase, b_per_w)])
        pltpu.sync_copy(tidx_hbm.at[pl.ds(pbase, b_per_w)], pidx_v)
        pltpu.async_copy(ent_hbm.at[pidx_v], prows_v, sem).wait()
        pltpu.sync_copy(prows_v, tout_hbm.at[pl.ds(pbase, b_per_w)])
        pltpu.sync_copy(ridx_hbm.at[pl.ds(pbase, b_per_w)], pidx_v)
        pltpu.async_copy(rel_hbm.at[pidx_v], prows_v, sem).wait()
        pltpu.sync_copy(prows_v, rout_hbm.at[pl.ds(pbase, b_per_w)])

    return k(entity_embedding, relation_embedding, neg_idx, head_idx,
             rel_idx, tail_idx)


def _normalize(x):
    return x * lax.rsqrt(jnp.sum(x * x, axis=-1, keepdims=True))


def _score_body(neg_ref, head_ref, tail_ref, rel_ref, cond_ref,
                pos_ref, negs_ref):
    h = head_ref[...]
    t = tail_ref[...]
    m = rel_ref[...]
    n = neg_ref[...]
    hd = h.shape[1] // 2

    ah = _normalize(h[:, :hd])
    bh = _normalize(h[:, hd:]) + U
    at = _normalize(t[:, :hd])
    bt = _normalize(t[:, hd:]) + U

    pos_ref[...] = GAMMA - jnp.sum(jnp.abs(ah * bt - at * bh + m),
                                   axis=-1, keepdims=True)

    an = _normalize(n[:, :, :hd])
    bn = _normalize(n[:, :, hd:]) + U
    m3 = m[:, None, :]
    tail_s = GAMMA - jnp.sum(
        jnp.abs(ah[:, None, :] * bn - an * bh[:, None, :] + m3), axis=-1)
    head_s = GAMMA - jnp.sum(
        jnp.abs(an * bt[:, None, :] - at[:, None, :] * bn + m3), axis=-1)
    c = cond_ref[0, 0]
    negs_ref[...] = head_s * c + tail_s * (1.0 - c)


def _tc_score(neg_rows, head_rows, tail_rows, rel_mid, cond):
    b, ent_dim = head_rows.shape
    neg = neg_rows.shape[0] // b
    hd = ent_dim // 2
    neg3 = neg_rows.reshape(b, neg, ent_dim)
    rb = 8
    grid = (b // rb,)
    return pl.pallas_call(
        _score_body,
        grid=grid,
        in_specs=[
            pl.BlockSpec((rb, neg, ent_dim), lambda i: (i, 0, 0)),
            pl.BlockSpec((rb, ent_dim), lambda i: (i, 0)),
            pl.BlockSpec((rb, ent_dim), lambda i: (i, 0)),
            pl.BlockSpec((rb, hd), lambda i: (i, 0)),
            pl.BlockSpec(memory_space=pltpu.SMEM),
        ],
        out_specs=[
            pl.BlockSpec((rb, 1), lambda i: (i, 0)),
            pl.BlockSpec((rb, neg), lambda i: (i, 0)),
        ],
        out_shape=[
            jax.ShapeDtypeStruct((b, 1), jnp.float32),
            jax.ShapeDtypeStruct((b, neg), jnp.float32),
        ],
    )(neg3, head_rows, tail_rows, rel_mid, cond)


def kernel(entity_embedding, relation_embedding, positive_sample,
           negative_sample, mode):
    neg_idx = negative_sample.reshape(-1).astype(jnp.int32)
    head_idx = positive_sample[:, 0].astype(jnp.int32)
    rel_idx = positive_sample[:, 1].astype(jnp.int32)
    tail_idx = positive_sample[:, 2].astype(jnp.int32)

    ent_dim = entity_embedding.shape[1]
    hd = ent_dim // 2
    neg_rows, head_rows, tail_rows, rel_rows = _sc_gather(
        entity_embedding, relation_embedding[:, :ent_dim], neg_idx,
        head_idx, rel_idx, tail_idx)

    cond = jnp.where(jnp.asarray(mode) == 0, 1.0, 0.0)
    cond = cond.astype(jnp.float32).reshape(1, 1)
    rel_mid = rel_rows[:, hd:2 * hd]
    positive_score, negative_score = _tc_score(
        neg_rows, head_rows, tail_rows, rel_mid, cond)
    return positive_score, negative_score
